# baseline (speedup 1.0000x reference)
import functools
import numpy as np
import jax
import jax.numpy as jnp
from jax import lax
from jax.experimental import pallas as pl
from jax.experimental.pallas import tpu as pltpu
from jax.experimental.pallas import tpu_sc as plsc

T = 26
B = 4096
L = 20
ROWS = 100000
DIMS = [32 if i % 2 == 0 else 64 for i in range(T)]
TOTAL_D = int(sum(DIMS))

NW = 32
BAGS_PER_W = B // NW
NB = 32
NCH = BAGS_PER_W // NB
RPC = NB * L
WC = NW * NCH
NU = TOTAL_D // 32
NK = NCH * NU

_mesh = plsc.VectorSubcoreMesh(core_axis_name="c", subcore_axis_name="s")


def _step_params(k, wid):
    c = k // NU
    u = k - c * NU
    wc = wid * NCH + c
    um = u % 3
    t = 2 * (u // 3) + jnp.where(um == 0, 0, 1)
    mult = jnp.where(um == 0, 1, 2)
    bias = 150000 * t - 50000 * (t % 2) + jnp.where(um == 2, 1, 0)
    return c, u, wc, t, mult, bias


@functools.partial(
    pl.kernel,
    out_type=jax.ShapeDtypeStruct((B * TOTAL_D,), jnp.float32),
    mesh=_mesh,
    compiler_params=pltpu.CompilerParams(use_tc_tiling_on_sc=False),
    scratch_types=[
        pltpu.VMEM((5, 128), jnp.int32),
        pltpu.VMEM((5, 128), jnp.int32),
        pltpu.VMEM((5, 128), jnp.int32),
        pltpu.VMEM((RPC,), jnp.float32),
        pltpu.VMEM((RPC,), jnp.float32),
        pltpu.VMEM((RPC,), jnp.float32),
        pltpu.VMEM((RPC, 32), jnp.float32),
        pltpu.VMEM((RPC, 32), jnp.float32),
        pltpu.VMEM((RPC, 32), jnp.float32),
        pltpu.VMEM((NB * TOTAL_D,), jnp.float32),
        pltpu.SemaphoreType.DMA,
        pltpu.SemaphoreType.DMA,
        pltpu.SemaphoreType.DMA,
        pltpu.SemaphoreType.DMA,
        pltpu.SemaphoreType.DMA,
        pltpu.SemaphoreType.DMA,
    ],
)
def _emb_kernel(wtab, idx4, psw4, out,
                idx_v0, idx_v1, idx_v2, psw_v0, psw_v1, psw_v2,
                rows_v0, rows_v1, rows_v2,
                outs_v, semg0, semg1, semg2, semio0, semio1, semio2):
    wid = lax.axis_index("s") * 2 + lax.axis_index("c")

    def fire_io(k, idx_v, psw_v, semio):
        _, _, wc, t, _, _ = _step_params(k, wid)
        pltpu.async_copy(idx4.at[t, wc], idx_v, semio)
        pltpu.async_copy(psw4.at[t, wc, 0], psw_v, semio)

    def launch(k, idx_v, psw_v, rows_v, semg, semio):
        _, _, wc, t, mult, bias = _step_params(k, wid)
        pltpu.make_async_copy(idx4.at[t, wc], idx_v, semio).wait()
        pltpu.make_async_copy(psw4.at[t, wc, 0], psw_v, semio).wait()
        mult_v = jnp.full((16,), mult, jnp.int32)
        bias_v = jnp.full((16,), bias, jnp.int32)
        for j in range(5):
            for q in range(8):
                sl = (j, pl.ds(q * 16, 16))
                idx_v[sl] = idx_v[sl] * mult_v + bias_v
        for j in range(5):
            pltpu.async_copy(wtab.at[idx_v.at[j]],
                             rows_v.at[pl.ds(j * 128, 128)], semg)

    def consume(k, idx_v, psw_v, rows_v, semg):
        c = k // NU
        u = k - c * NU
        for j in range(5):
            pltpu.make_async_copy(wtab.at[idx_v.at[j]],
                                  rows_v.at[pl.ds(j * 128, 128)], semg).wait()
        colbase = u * 32

        @pl.loop(0, NB)
        def _(b):
            r0 = b * L
            w0 = psw_v[pl.ds(r0, 16)]
            w1 = psw_v[pl.ds(r0 + 4, 16)]
            accs = [jnp.zeros((16,), jnp.float32) for _ in range(4)]
            for l in range(L):
                s = w0[l] if l < 16 else w1[l - 4]
                w = jnp.full((16,), s, jnp.float32)
                for d in range(2):
                    a = 2 * d + (l & 1)
                    accs[a] = accs[a] + w * rows_v[r0 + l, pl.ds(d * 16, 16)]
            ob = b * TOTAL_D + colbase
            for d in range(2):
                outs_v[pl.ds(ob + d * 16, 16)] = accs[2 * d] + accs[2 * d + 1]

        @pl.when(u == NU - 1)
        def _():
            base = (wid * BAGS_PER_W + c * NB) * TOTAL_D
            pltpu.sync_copy(outs_v, out.at[pl.ds(base, NB * TOTAL_D)])

    bufs = [
        (idx_v0, psw_v0, rows_v0, semg0, semio0),
        (idx_v1, psw_v1, rows_v1, semg1, semio1),
        (idx_v2, psw_v2, rows_v2, semg2, semio2),
    ]

    def io_of(bf):
        return bf[0], bf[1], bf[4]

    def gather_of(bf):
        return bf[0], bf[1], bf[2], bf[3], bf[4]

    def cons_of(bf):
        return bf[0], bf[1], bf[2], bf[3]

    fire_io(0, *io_of(bufs[0]))
    launch(0, *gather_of(bufs[0]))
    fire_io(1, *io_of(bufs[1]))
    fire_io(2, *io_of(bufs[2]))

    @pl.loop(0, NK // 3)
    def _(i):
        k0 = 3 * i
        for p in range(3):
            k = k0 + p
            bnext = bufs[(p + 1) % 3]
            bcur = bufs[p]

            @pl.when(k + 1 < NK)
            def _():
                launch(k + 1, *gather_of(bnext))

            consume(k, *cons_of(bcur))

            @pl.when(k + 3 < NK)
            def _():
                fire_io(k + 3, *io_of(bcur))


def kernel(weights, sharded_sparse_features, sharded_offsets, per_sample_weights):
    del sharded_offsets
    idx4 = sharded_sparse_features.astype(jnp.int32).reshape(T, WC, 5, 128)
    psw4 = per_sample_weights.reshape(T, WC, 1, RPC)
    wtab = weights.reshape(-1, 32)
    return _emb_kernel(wtab, idx4, psw4).reshape(B, TOTAL_D)


# speedup vs baseline: 1.0098x; 1.0098x over previous
import functools
import numpy as np
import jax
import jax.numpy as jnp
from jax import lax
from jax.experimental import pallas as pl
from jax.experimental.pallas import tpu as pltpu
from jax.experimental.pallas import tpu_sc as plsc

T = 26
B = 4096
L = 20
ROWS = 100000
DIMS = [32 if i % 2 == 0 else 64 for i in range(T)]
TOTAL_D = int(sum(DIMS))

NW = 32
BAGS_PER_W = B // NW
NB = 32
NCH = BAGS_PER_W // NB
RPC = NB * L
WC = NW * NCH
NU = TOTAL_D // 32
NK = NCH * NU

_mesh = plsc.VectorSubcoreMesh(core_axis_name="c", subcore_axis_name="s")


def _step_params(k, wid):
    c = k // NU
    u = k - c * NU
    wc = wid * NCH + c
    um = u % 3
    t = 2 * (u // 3) + jnp.where(um == 0, 0, 1)
    mult = jnp.where(um == 0, 1, 2)
    bias = 150000 * t - 50000 * (t % 2) + jnp.where(um == 2, 1, 0)
    return c, u, wc, t, mult, bias


@functools.partial(
    pl.kernel,
    out_type=jax.ShapeDtypeStruct((B * TOTAL_D,), jnp.float32),
    mesh=_mesh,
    compiler_params=pltpu.CompilerParams(use_tc_tiling_on_sc=False),
    scratch_types=[
        pltpu.VMEM((RPC,), jnp.int32),
        pltpu.VMEM((RPC,), jnp.int32),
        pltpu.VMEM((RPC,), jnp.int32),
        pltpu.VMEM((RPC,), jnp.float32),
        pltpu.VMEM((RPC,), jnp.float32),
        pltpu.VMEM((RPC,), jnp.float32),
        pltpu.VMEM((RPC, 32), jnp.float32),
        pltpu.VMEM((RPC, 32), jnp.float32),
        pltpu.VMEM((RPC, 32), jnp.float32),
        pltpu.VMEM((NB * TOTAL_D,), jnp.float32),
        pltpu.SemaphoreType.DMA,
        pltpu.SemaphoreType.DMA,
        pltpu.SemaphoreType.DMA,
        pltpu.SemaphoreType.DMA,
        pltpu.SemaphoreType.DMA,
        pltpu.SemaphoreType.DMA,
    ],
)
def _emb_kernel(wtab, idx4, psw4, out,
                idx_v0, idx_v1, idx_v2, psw_v0, psw_v1, psw_v2,
                rows_v0, rows_v1, rows_v2,
                outs_v, semg0, semg1, semg2, semio0, semio1, semio2):
    wid = lax.axis_index("s") * 2 + lax.axis_index("c")

    def fire_io(k, idx_v, psw_v, semio):
        _, _, wc, t, _, _ = _step_params(k, wid)
        pltpu.async_copy(idx4.at[t, wc, 0], idx_v, semio)
        pltpu.async_copy(psw4.at[t, wc, 0], psw_v, semio)

    def launch(k, idx_v, psw_v, rows_v, semg, semio):
        _, _, wc, t, mult, bias = _step_params(k, wid)
        pltpu.make_async_copy(idx4.at[t, wc, 0], idx_v, semio).wait()
        pltpu.make_async_copy(psw4.at[t, wc, 0], psw_v, semio).wait()
        mult_v = jnp.full((16,), mult, jnp.int32)
        bias_v = jnp.full((16,), bias, jnp.int32)
        for q in range(RPC // 16):
            sl = pl.ds(q * 16, 16)
            idx_v[sl] = idx_v[sl] * mult_v + bias_v
        pltpu.async_copy(wtab.at[idx_v], rows_v, semg)

    def consume(k, idx_v, psw_v, rows_v, semg):
        c = k // NU
        u = k - c * NU
        pltpu.make_async_copy(wtab.at[idx_v], rows_v, semg).wait()
        colbase = u * 32

        @pl.loop(0, NB)
        def _(b):
            r0 = b * L
            w0 = psw_v[pl.ds(r0, 16)]
            w1 = psw_v[pl.ds(r0 + 4, 16)]
            accs = [jnp.zeros((16,), jnp.float32) for _ in range(2)]
            for l in range(L):
                s = w0[l] if l < 16 else w1[l - 4]
                w = jnp.full((16,), s, jnp.float32)
                for d in range(2):
                    accs[d] = accs[d] + w * rows_v[r0 + l, pl.ds(d * 16, 16)]
            ob = b * TOTAL_D + colbase
            for d in range(2):
                outs_v[pl.ds(ob + d * 16, 16)] = accs[d]

        @pl.when(u == NU - 1)
        def _():
            base = (wid * BAGS_PER_W + c * NB) * TOTAL_D
            pltpu.sync_copy(outs_v, out.at[pl.ds(base, NB * TOTAL_D)])

    bufs = [
        (idx_v0, psw_v0, rows_v0, semg0, semio0),
        (idx_v1, psw_v1, rows_v1, semg1, semio1),
        (idx_v2, psw_v2, rows_v2, semg2, semio2),
    ]

    def io_of(bf):
        return bf[0], bf[1], bf[4]

    def gather_of(bf):
        return bf[0], bf[1], bf[2], bf[3], bf[4]

    def cons_of(bf):
        return bf[0], bf[1], bf[2], bf[3]

    fire_io(0, *io_of(bufs[0]))
    launch(0, *gather_of(bufs[0]))
    fire_io(1, *io_of(bufs[1]))
    fire_io(2, *io_of(bufs[2]))

    @pl.loop(0, NK // 3)
    def _(i):
        k0 = 3 * i
        for p in range(3):
            k = k0 + p
            bnext = bufs[(p + 1) % 3]
            bcur = bufs[p]

            @pl.when(k + 1 < NK)
            def _():
                launch(k + 1, *gather_of(bnext))

            consume(k, *cons_of(bcur))

            @pl.when(k + 3 < NK)
            def _():
                fire_io(k + 3, *io_of(bcur))


def kernel(weights, sharded_sparse_features, sharded_offsets, per_sample_weights):
    del sharded_offsets
    idx4 = sharded_sparse_features.astype(jnp.int32).reshape(T, WC, 1, RPC)
    psw4 = per_sample_weights.reshape(T, WC, 1, RPC)
    wtab = weights.reshape(-1, 32)
    return _emb_kernel(wtab, idx4, psw4).reshape(B, TOTAL_D)


# speedup vs baseline: 1.1827x; 1.1711x over previous
import functools
import numpy as np
import jax
import jax.numpy as jnp
from jax import lax
from jax.experimental import pallas as pl
from jax.experimental.pallas import tpu as pltpu
from jax.experimental.pallas import tpu_sc as plsc

T = 26
B = 4096
L = 20
ROWS = 100000
DIMS = [32 if i % 2 == 0 else 64 for i in range(T)]
TOTAL_D = int(sum(DIMS))

NW = 32
BAGS_PER_W = B // NW
NB = 32
NCH = BAGS_PER_W // NB
RPC = NB * L
WC = NW * NCH
NU = TOTAL_D // 32
NK = NCH * NU

_mesh = plsc.VectorSubcoreMesh(core_axis_name="c", subcore_axis_name="s")


def _step_params(k, wid):
    c = k // NU
    u = k - c * NU
    wc = wid * NCH + c
    um = u % 3
    t = 2 * (u // 3) + jnp.where(um == 0, 0, 1)
    mult = jnp.where(um == 0, 1, 2)
    bias = 150000 * t - 50000 * (t % 2) + jnp.where(um == 2, 1, 0)
    return c, u, wc, t, mult, bias


@functools.partial(
    pl.kernel,
    out_type=jax.ShapeDtypeStruct((B * TOTAL_D,), jnp.float32),
    mesh=_mesh,
    compiler_params=pltpu.CompilerParams(use_tc_tiling_on_sc=False),
    scratch_types=[
        pltpu.VMEM((RPC,), jnp.int32),
        pltpu.VMEM((RPC,), jnp.int32),
        pltpu.VMEM((RPC,), jnp.int32),
        pltpu.VMEM((RPC,), jnp.int32),
        pltpu.VMEM((RPC,), jnp.float32),
        pltpu.VMEM((RPC,), jnp.float32),
        pltpu.VMEM((RPC,), jnp.float32),
        pltpu.VMEM((RPC,), jnp.float32),
        pltpu.VMEM((RPC, 32), jnp.float32),
        pltpu.VMEM((RPC, 32), jnp.float32),
        pltpu.VMEM((RPC, 32), jnp.float32),
        pltpu.VMEM((RPC, 32), jnp.float32),
        pltpu.VMEM((NB * TOTAL_D,), jnp.float32),
        pltpu.SemaphoreType.DMA,
        pltpu.SemaphoreType.DMA,
        pltpu.SemaphoreType.DMA,
        pltpu.SemaphoreType.DMA,
        pltpu.SemaphoreType.DMA,
        pltpu.SemaphoreType.DMA,
        pltpu.SemaphoreType.DMA,
        pltpu.SemaphoreType.DMA,
    ],
)
def _emb_kernel(wtab, idx4, psw4, out,
                idx_v0, idx_v1, idx_v2, idx_v3,
                psw_v0, psw_v1, psw_v2, psw_v3,
                rows_v0, rows_v1, rows_v2, rows_v3,
                outs_v, semg0, semg1, semg2, semg3,
                semio0, semio1, semio2, semio3):
    wid = lax.axis_index("s") * 2 + lax.axis_index("c")

    def fire_io(k, idx_v, psw_v, semio):
        _, _, wc, t, _, _ = _step_params(k, wid)
        pltpu.async_copy(idx4.at[t, wc, 0], idx_v, semio)
        pltpu.async_copy(psw4.at[t, wc, 0], psw_v, semio)

    def launch(k, idx_v, psw_v, rows_v, semg, semio):
        _, _, wc, t, mult, bias = _step_params(k, wid)
        pltpu.make_async_copy(idx4.at[t, wc, 0], idx_v, semio).wait()
        pltpu.make_async_copy(psw4.at[t, wc, 0], psw_v, semio).wait()
        mult_v = jnp.full((16,), mult, jnp.int32)
        bias_v = jnp.full((16,), bias, jnp.int32)
        for q in range(RPC // 16):
            sl = pl.ds(q * 16, 16)
            idx_v[sl] = idx_v[sl] * mult_v + bias_v
        pltpu.async_copy(wtab.at[idx_v], rows_v, semg)

    def consume(k, idx_v, psw_v, rows_v, semg):
        c = k // NU
        u = k - c * NU
        pltpu.make_async_copy(wtab.at[idx_v], rows_v, semg).wait()
        colbase = u * 32

        @pl.loop(0, NB)
        def _(b):
            r0 = b * L
            w0 = psw_v[pl.ds(r0, 16)]
            w1 = psw_v[pl.ds(r0 + 4, 16)]
            accs = [jnp.zeros((16,), jnp.float32) for _ in range(2)]
            for l in range(L):
                s = w0[l] if l < 16 else w1[l - 4]
                w = jnp.full((16,), s, jnp.float32)
                for d in range(2):
                    accs[d] = accs[d] + w * rows_v[r0 + l, pl.ds(d * 16, 16)]
            ob = b * TOTAL_D + colbase
            for d in range(2):
                outs_v[pl.ds(ob + d * 16, 16)] = accs[d]

        @pl.when(u == NU - 1)
        def _():
            base = (wid * BAGS_PER_W + c * NB) * TOTAL_D
            pltpu.sync_copy(outs_v, out.at[pl.ds(base, NB * TOTAL_D)])

    bufs = [
        (idx_v0, psw_v0, rows_v0, semg0, semio0),
        (idx_v1, psw_v1, rows_v1, semg1, semio1),
        (idx_v2, psw_v2, rows_v2, semg2, semio2),
        (idx_v3, psw_v3, rows_v3, semg3, semio3),
    ]

    def io_of(bf):
        return bf[0], bf[1], bf[4]

    def gather_of(bf):
        return bf[0], bf[1], bf[2], bf[3], bf[4]

    def cons_of(bf):
        return bf[0], bf[1], bf[2], bf[3]

    fire_io(0, *io_of(bufs[0]))
    launch(0, *gather_of(bufs[0]))
    fire_io(1, *io_of(bufs[1]))
    fire_io(2, *io_of(bufs[2]))
    launch(1, *gather_of(bufs[1]))
    fire_io(3, *io_of(bufs[3]))

    @pl.loop(0, NK // 4)
    def _(i):
        k0 = 4 * i
        for p in range(4):
            k = k0 + p
            bnext2 = bufs[(p + 2) % 4]
            bcur = bufs[p]

            @pl.when(k + 2 < NK)
            def _():
                launch(k + 2, *gather_of(bnext2))

            consume(k, *cons_of(bcur))

            @pl.when(k + 4 < NK)
            def _():
                fire_io(k + 4, *io_of(bcur))


def kernel(weights, sharded_sparse_features, sharded_offsets, per_sample_weights):
    del sharded_offsets
    idx4 = sharded_sparse_features.astype(jnp.int32).reshape(T, WC, 1, RPC)
    psw4 = per_sample_weights.reshape(T, WC, 1, RPC)
    wtab = weights.reshape(-1, 32)
    return _emb_kernel(wtab, idx4, psw4).reshape(B, TOTAL_D)
